# Initial kernel scaffold; baseline (speedup 1.0000x reference)
import jax
import jax.numpy as jnp
from jax.experimental import pallas as pl


def kernel(x, codebook):
    raise NotImplementedError("write your pallas kernel here")


# speedup vs baseline: 1.1572x; 1.1572x over previous
import jax
import jax.numpy as jnp
from jax.experimental import pallas as pl


_BN = 256
_W = 4096


def _vq_argmin_kernel(x_ref, xn_ref, cb_ref, cn_ref, out_ref):
    xb = x_ref[...].astype(jnp.bfloat16)
    cbb = cb_ref[...].astype(jnp.bfloat16)
    m = jax.lax.dot_general(
        xb, cbb, (((1,), (1,)), ((), ())), preferred_element_type=jnp.float32
    )
    d = (xn_ref[...] + cn_ref[...]) - 2.0 * m
    bn, k = d.shape

    run_v = jnp.full((bn, 1), jnp.inf, jnp.float32)
    run_i = jnp.zeros((bn, 1), jnp.int32)
    for w in range(k // _W):
        blk = jax.lax.slice(d, (0, w * _W), (bn, (w + 1) * _W))
        vmin = jnp.min(blk, axis=1, keepdims=True)
        iota = jax.lax.broadcasted_iota(jnp.int32, blk.shape, 1)
        imin = jnp.min(jnp.where(blk == vmin, iota + w * _W, k),
                       axis=1, keepdims=True)
        take = vmin < run_v
        run_i = jnp.where(take, imin, run_i)
        run_v = jnp.where(take, vmin, run_v).astype(jnp.bfloat16)
        run_v = run_v.astype(jnp.float32)
    out_ref[...] = run_i


def kernel(x, codebook):
    B = x.shape[0]
    code_dim = codebook.shape[1]
    K = codebook.shape[0]
    flattened = x.reshape(-1, code_dim)
    N = flattened.shape[0]
    xnorm = jnp.sum(flattened ** 2, axis=1, keepdims=True)
    cnorm = jnp.sum(codebook ** 2, axis=1)[None, :]

    codes = pl.pallas_call(
        _vq_argmin_kernel,
        grid=(N // _BN,),
        in_specs=[
            pl.BlockSpec((_BN, code_dim), lambda i: (i, 0)),
            pl.BlockSpec((_BN, 1), lambda i: (i, 0)),
            pl.BlockSpec((K, code_dim), lambda i: (0, 0)),
            pl.BlockSpec((1, K), lambda i: (0, 0)),
        ],
        out_specs=pl.BlockSpec((_BN, 1), lambda i: (i, 0)),
        out_shape=jax.ShapeDtypeStruct((N, 1), jnp.int32),
    )(flattened, xnorm, codebook, cnorm)
    return codes.reshape(B, -1)
